# Initial kernel scaffold; baseline (speedup 1.0000x reference)
import jax
import jax.numpy as jnp
from jax.experimental import pallas as pl


def kernel(x, edge_index, W1, b1, W2, b2, Wc1, bc1, Wc2, bc2, Wp, bp):
    raise NotImplementedError("write your pallas kernel here")


# speedup vs baseline: 10.7676x; 10.7676x over previous
import functools

import jax
import jax.numpy as jnp
from jax import lax
from jax.experimental import pallas as pl
from jax.experimental.pallas import tpu as pltpu
from jax.experimental.pallas import tpu_sc as plsc

N = 10000
E = 320000
D = 128

NC = 2
NS = 16
NW = NC * NS
CHUNK = 128
NCHUNK = 79
EPW = CHUNK * NCHUNK
E_PAD = EPW * NW
ACC_ROWS = 10240
ROWS_PER_TILE = ACC_ROWS // NS


def _leaky(h):
    return jnp.where(h >= 0, h, 0.01 * h)


def _deg_body(dst_hbm, ones_hbm, zero_hbm, out_hbm, idx_v, ones_v, acc_sh, sem):
    c = lax.axis_index("c")
    s = lax.axis_index("s")
    wid = c * NS + s
    pltpu.sync_copy(zero_hbm, acc_sh.at[pl.ds(s * ROWS_PER_TILE, ROWS_PER_TILE)])
    pltpu.sync_copy(ones_hbm, ones_v)
    plsc.subcore_barrier()

    def body(t, _):
        base = wid * EPW + t * CHUNK
        pltpu.sync_copy(dst_hbm.at[pl.ds(base, CHUNK)], idx_v)
        pltpu.sync_copy(ones_v, acc_sh.at[idx_v], add=True)
        return 0

    lax.fori_loop(0, NCHUNK, body, 0)
    plsc.subcore_barrier()
    pltpu.sync_copy(
        acc_sh.at[pl.ds(s * ROWS_PER_TILE, ROWS_PER_TILE)],
        out_hbm.at[c, pl.ds(s * ROWS_PER_TILE, ROWS_PER_TILE)],
    )


@functools.cache
def _deg_kernel():
    mesh = plsc.VectorSubcoreMesh(
        core_axis_name="c", subcore_axis_name="s",
        num_cores=NC, num_subcores=NS,
    )
    return pl.kernel(
        _deg_body,
        out_type=jax.ShapeDtypeStruct((NC, ACC_ROWS, D), jnp.float32),
        mesh=mesh,
        scratch_types=[
            pltpu.VMEM((CHUNK,), jnp.int32),
            pltpu.VMEM((CHUNK, D), jnp.float32),
            pltpu.VMEM_SHARED((ACC_ROWS, D), jnp.float32),
            pltpu.SemaphoreType.DMA,
        ],
    )


def _agg_body(g_hbm, src_hbm, dst_hbm, zrows_hbm, out_hbm,
              src_v, dst_v, rows_v, acc_sh, sem):
    c = lax.axis_index("c")
    s = lax.axis_index("s")
    wid = c * NS + s
    pltpu.sync_copy(zrows_hbm, acc_sh.at[pl.ds(s * ROWS_PER_TILE, ROWS_PER_TILE)])
    plsc.subcore_barrier()

    def body(t, _):
        base = wid * EPW + t * CHUNK
        pltpu.sync_copy(src_hbm.at[pl.ds(base, CHUNK)], src_v)
        pltpu.sync_copy(dst_hbm.at[pl.ds(base, CHUNK)], dst_v)
        pltpu.async_copy(g_hbm.at[src_v], rows_v, sem).wait()
        pltpu.sync_copy(rows_v, acc_sh.at[dst_v], add=True)
        return 0

    lax.fori_loop(0, NCHUNK, body, 0)
    plsc.subcore_barrier()
    pltpu.sync_copy(
        acc_sh.at[pl.ds(s * ROWS_PER_TILE, ROWS_PER_TILE)],
        out_hbm.at[c, pl.ds(s * ROWS_PER_TILE, ROWS_PER_TILE)],
    )


@functools.cache
def _agg_kernel():
    mesh = plsc.VectorSubcoreMesh(
        core_axis_name="c", subcore_axis_name="s",
        num_cores=NC, num_subcores=NS,
    )
    return pl.kernel(
        _agg_body,
        out_type=jax.ShapeDtypeStruct((NC, ACC_ROWS, D), jnp.float32),
        mesh=mesh,
        scratch_types=[
            pltpu.VMEM((CHUNK,), jnp.int32),
            pltpu.VMEM((CHUNK,), jnp.int32),
            pltpu.VMEM((CHUNK, D), jnp.float32),
            pltpu.VMEM_SHARED((ACC_ROWS, D), jnp.float32),
            pltpu.SemaphoreType.DMA,
        ],
    )


def _tc1_body(x_ref, w1_ref, b1_ref, w2_ref, b2_ref, h_ref):
    h = _leaky(jnp.dot(x_ref[...], w1_ref[...],
                       preferred_element_type=jnp.float32) + b1_ref[...])
    h_ref[...] = _leaky(jnp.dot(h, w2_ref[...],
                                preferred_element_type=jnp.float32) + b2_ref[...])


def _tc2_body(degp_ref, h_ref, wc_ref, g_ref, dinv_ref):
    deg = degp_ref[0, :N, 0:1] + degp_ref[1, :N, 0:1] + 1.0
    dinv = lax.rsqrt(deg)
    dinv_ref[...] = dinv
    g_ref[...] = dinv * jnp.dot(h_ref[...], wc_ref[...],
                                preferred_element_type=jnp.float32)


def _tc3_body(s_ref, g_ref, dinv_ref, bc_ref, wc_ref, emb_ref, g2_ref):
    ssum = s_ref[0, :N, :] + s_ref[1, :N, :] + g_ref[...]
    dinv = dinv_ref[...]
    emb = _leaky(dinv * ssum + bc_ref[...])
    emb_ref[...] = emb
    g2_ref[...] = dinv * jnp.dot(emb, wc_ref[...],
                                 preferred_element_type=jnp.float32)


def _tc4_body(s_ref, g_ref, dinv_ref, bc_ref, wp_ref, bp_ref, emb_ref, o_ref):
    ssum = s_ref[0, :N, :] + s_ref[1, :N, :] + g_ref[...]
    emb = _leaky(dinv_ref[...] * ssum + bc_ref[...])
    emb_ref[...] = emb
    o_ref[...] = jnp.dot(emb, wp_ref[...],
                         preferred_element_type=jnp.float32) + bp_ref[...]


_f32 = jnp.float32


def _tc1(x, W1, b1, W2, b2):
    return pl.pallas_call(
        _tc1_body, out_shape=jax.ShapeDtypeStruct((N, D), _f32)
    )(x, W1, b1, W2, b2)


def _tc2(degp, h, Wc):
    return pl.pallas_call(
        _tc2_body,
        out_shape=(jax.ShapeDtypeStruct((N, D), _f32),
                   jax.ShapeDtypeStruct((N, 1), _f32)),
    )(degp, h, Wc)


def _tc3(S, g, dinv, bc, Wc):
    return pl.pallas_call(
        _tc3_body,
        out_shape=(jax.ShapeDtypeStruct((N, D), _f32),
                   jax.ShapeDtypeStruct((N, D), _f32)),
    )(S, g, dinv, bc, Wc)


def _tc4(S, g, dinv, bc, Wp, bp):
    return pl.pallas_call(
        _tc4_body,
        out_shape=(jax.ShapeDtypeStruct((N, D), _f32),
                   jax.ShapeDtypeStruct((N, 1), _f32)),
    )(S, g, dinv, bc, Wp, bp)


@jax.jit
def kernel(x, edge_index, W1, b1, W2, b2, Wc1, bc1, Wc2, bc2, Wp, bp):
    pad = E_PAD - E
    src = jnp.concatenate([edge_index[0], jnp.zeros((pad,), jnp.int32)])
    dst = jnp.concatenate([edge_index[1], jnp.full((pad,), N, jnp.int32)])
    ones128 = jnp.ones((CHUNK, D), _f32)
    zrows = jnp.zeros((ROWS_PER_TILE, D), _f32)

    h = _tc1(x, W1, b1.reshape(1, D), W2, b2.reshape(1, D))
    degp = _deg_kernel()(dst, ones128, zrows)
    g1, dinv = _tc2(degp, h, Wc1)
    S1 = _agg_kernel()(g1, src, dst, zrows)
    emb1, g2 = _tc3(S1, g1, dinv, bc1.reshape(1, D), Wc2)
    S2 = _agg_kernel()(g2, src, dst, zrows)
    emb2, o = _tc4(S2, g2, dinv, bc2.reshape(1, D), Wp, bp.reshape(1, 1))
    return (o.reshape(N), emb1, emb2)
